# baseline (device time: 34507 ns/iter reference)
import jax
import jax.numpy as jnp
from jax import lax
from jax.experimental import pallas as pl
from jax.experimental.pallas import tpu as pltpu

N_DEV = 8


def kernel(x, Win0, Wout0, Win1, Wout1, Win2, Wout2):
    b, d_model = x.shape
    _, h_sh = Win0.shape
    rows = b // N_DEV

    def body(x_hbm, win0_hbm, wout0_hbm, win1_hbm, wout1_hbm, win2_hbm,
             wout2_hbm, out_ref, x_v, win_v, wout_v, part_ref, rs_ref,
             ag_ref, load_sems,
             rs_send_sems, rs_recv_sems, ag_send_sems, ag_recv_sems):
        my = lax.axis_index("i")

        loads = []
        for i, (src, dst) in enumerate([
            (x_hbm, x_v),
            (win0_hbm, win_v.at[0]), (wout0_hbm, wout_v.at[0]),
            (win1_hbm, win_v.at[1]), (wout1_hbm, wout_v.at[1]),
            (win2_hbm, win_v.at[2]), (wout2_hbm, wout_v.at[2]),
        ]):
            cp = pltpu.make_async_copy(src, dst, load_sems.at[i])
            cp.start()
            loads.append(cp)

        def layer_partial(x_bf16, l):
            w_in = win_v[l].astype(jnp.bfloat16)
            w_out = wout_v[l].astype(jnp.bfloat16)
            h = jnp.dot(x_bf16, w_in, preferred_element_type=jnp.float32)
            h = jnp.maximum(h, 0.0).astype(jnp.bfloat16)
            return jnp.dot(h, w_out, preferred_element_type=jnp.float32)

        def rs_send(chunk_idx):
            rdma = pltpu.make_async_remote_copy(
                src_ref=part_ref.at[pl.ds(chunk_idx * rows, rows), :],
                dst_ref=rs_ref.at[my],
                send_sem=rs_send_sems.at[chunk_idx],
                recv_sem=rs_recv_sems.at[my],
                device_id=(chunk_idx,),
                device_id_type=pl.DeviceIdType.MESH,
            )
            rdma.start()
            return rdma

        def rs_finish(sends):
            for off in range(1, N_DEV):
                s = lax.rem(my + off, N_DEV)
                recv = pltpu.make_async_remote_copy(
                    src_ref=rs_ref.at[s],
                    dst_ref=rs_ref.at[s],
                    send_sem=rs_send_sems.at[s],
                    recv_sem=rs_recv_sems.at[s],
                    device_id=(s,),
                    device_id_type=pl.DeviceIdType.MESH,
                )
                recv.wait_recv()
            reduced = jnp.sum(rs_ref[...].astype(jnp.float32), axis=0)
            for rdma in sends:
                rdma.wait_send()
            return reduced

        for cp in loads[:3]:
            cp.wait()
        part_ref[...] = layer_partial(
            x_v[...].astype(jnp.bfloat16), 0
        ).astype(jnp.bfloat16)
        sends = [rs_send(lax.rem(my + off, N_DEV)) for off in range(1, N_DEV)]
        rs_ref[my] = part_ref[pl.ds(my * rows, rows), :]
        reduced = rs_finish(sends)

        for l in (1, 2):
            ag_ref[my] = reduced.astype(jnp.bfloat16)
            ag_sends = []
            for off in range(1, N_DEV):
                d = lax.rem(my + off, N_DEV)
                rdma = pltpu.make_async_remote_copy(
                    src_ref=ag_ref.at[my],
                    dst_ref=ag_ref.at[my],
                    send_sem=ag_send_sems.at[d],
                    recv_sem=ag_recv_sems.at[my],
                    device_id=(d,),
                    device_id_type=pl.DeviceIdType.MESH,
                )
                rdma.start()
                ag_sends.append(rdma)

            for cp in loads[2 * l + 1:2 * l + 3]:
                cp.wait()

            rs_ref[my] = layer_partial(ag_ref[my], l).astype(jnp.bfloat16)

            sends = []
            for off in range(1, N_DEV):
                s = lax.rem(my + off, N_DEV)
                recv = pltpu.make_async_remote_copy(
                    src_ref=ag_ref.at[s],
                    dst_ref=ag_ref.at[s],
                    send_sem=ag_send_sems.at[s],
                    recv_sem=ag_recv_sems.at[s],
                    device_id=(s,),
                    device_id_type=pl.DeviceIdType.MESH,
                )
                recv.wait_recv()
                part_ref[pl.ds(s * rows, rows), :] = layer_partial(
                    ag_ref[s], l
                ).astype(jnp.bfloat16)
                sends.append(rs_send(s))
            reduced = rs_finish(sends)
            for rdma in ag_sends:
                rdma.wait_send()

        out_ref[...] = reduced

    out_shape = jax.ShapeDtypeStruct((rows, d_model), jnp.float32)
    anyspec = pl.BlockSpec(memory_space=pl.ANY)
    return pl.pallas_call(
        body,
        out_shape=out_shape,
        in_specs=[anyspec] * 7,
        out_specs=pl.BlockSpec(memory_space=pltpu.VMEM),
        scratch_shapes=[
            pltpu.VMEM((b, d_model), jnp.float32),
            pltpu.VMEM((3, d_model, h_sh), jnp.float32),
            pltpu.VMEM((3, h_sh, d_model), jnp.float32),
            pltpu.VMEM((b, d_model), jnp.bfloat16),
            pltpu.VMEM((N_DEV, rows, d_model), jnp.bfloat16),
            pltpu.VMEM((N_DEV, rows, d_model), jnp.bfloat16),
            pltpu.SemaphoreType.DMA((7,)),
            pltpu.SemaphoreType.DMA((N_DEV,)),
            pltpu.SemaphoreType.DMA((N_DEV,)),
            pltpu.SemaphoreType.DMA((N_DEV,)),
            pltpu.SemaphoreType.DMA((N_DEV,)),
        ],
    )(x, Win0, Wout0, Win1, Wout1, Win2, Wout2)


# device time: 34065 ns/iter; 1.0130x vs baseline; 1.0130x over previous
import jax
import jax.numpy as jnp
from jax import lax
from jax.experimental import pallas as pl
from jax.experimental.pallas import tpu as pltpu

N_DEV = 8
MASKS_FAR_FIRST = (7, 6, 5, 3, 4, 2, 1)
MASKS_NEAR_FIRST = tuple(reversed(MASKS_FAR_FIRST))


def kernel(x, Win0, Wout0, Win1, Wout1, Win2, Wout2):
    b, d_model = x.shape
    rows = b // N_DEV

    def body(x_ref, win0_ref, wout0_ref, win1_ref, wout1_ref, win2_ref,
             wout2_ref, out_ref, part_ref, rs_ref, ag_ref,
             rs_send_sems, rs_recv_sems, ag_send_sems, ag_recv_sems):
        my = lax.axis_index("i")

        def gray2(t):
            return t ^ (t >> 1)

        my_bits = (my & 4) | gray2(my & 3)

        def peer_at(mask):
            pb = my_bits ^ mask
            return (pb & 4) | gray2(pb & 3)

        def layer_partial(x_bf16, win_ref, wout_ref):
            w_in = win_ref[...].astype(jnp.bfloat16)
            w_out = wout_ref[...].astype(jnp.bfloat16)
            h = jnp.dot(x_bf16, w_in, preferred_element_type=jnp.float32)
            h = jnp.maximum(h, 0.0).astype(jnp.bfloat16)
            return jnp.dot(h, w_out, preferred_element_type=jnp.float32)

        def rs_send(chunk_idx):
            rdma = pltpu.make_async_remote_copy(
                src_ref=part_ref.at[pl.ds(chunk_idx * rows, rows), :],
                dst_ref=rs_ref.at[my],
                send_sem=rs_send_sems.at[chunk_idx],
                recv_sem=rs_recv_sems.at[my],
                device_id=(chunk_idx,),
                device_id_type=pl.DeviceIdType.MESH,
            )
            rdma.start()
            return rdma

        def rs_finish(sends):
            acc = rs_ref[my].astype(jnp.float32)
            for m in MASKS_NEAR_FIRST:
                s = peer_at(m)
                recv = pltpu.make_async_remote_copy(
                    src_ref=rs_ref.at[s],
                    dst_ref=rs_ref.at[s],
                    send_sem=rs_send_sems.at[s],
                    recv_sem=rs_recv_sems.at[s],
                    device_id=(s,),
                    device_id_type=pl.DeviceIdType.MESH,
                )
                recv.wait_recv()
                acc = acc + rs_ref[s].astype(jnp.float32)
            for rdma in sends:
                rdma.wait_send()
            return acc

        part_ref[...] = layer_partial(
            x_ref[...].astype(jnp.bfloat16), win0_ref, wout0_ref
        ).astype(jnp.bfloat16)
        rs_ref[my] = part_ref[pl.ds(my * rows, rows), :]
        sends = [rs_send(peer_at(m)) for m in MASKS_FAR_FIRST]
        reduced = rs_finish(sends)

        for win_ref, wout_ref in [(win1_ref, wout1_ref), (win2_ref, wout2_ref)]:
            ag_ref[my] = reduced.astype(jnp.bfloat16)
            ag_sends = []
            for m in MASKS_FAR_FIRST:
                d = peer_at(m)
                rdma = pltpu.make_async_remote_copy(
                    src_ref=ag_ref.at[my],
                    dst_ref=ag_ref.at[my],
                    send_sem=ag_send_sems.at[d],
                    recv_sem=ag_recv_sems.at[my],
                    device_id=(d,),
                    device_id_type=pl.DeviceIdType.MESH,
                )
                rdma.start()
                ag_sends.append(rdma)

            rs_ref[my] = layer_partial(
                ag_ref[my], win_ref, wout_ref
            ).astype(jnp.bfloat16)

            sends = []
            for m in MASKS_NEAR_FIRST:
                s = peer_at(m)
                recv = pltpu.make_async_remote_copy(
                    src_ref=ag_ref.at[s],
                    dst_ref=ag_ref.at[s],
                    send_sem=ag_send_sems.at[s],
                    recv_sem=ag_recv_sems.at[s],
                    device_id=(s,),
                    device_id_type=pl.DeviceIdType.MESH,
                )
                recv.wait_recv()
                part_ref[pl.ds(s * rows, rows), :] = layer_partial(
                    ag_ref[s], win_ref, wout_ref
                ).astype(jnp.bfloat16)
                sends.append(rs_send(s))
            reduced = rs_finish(sends)
            for rdma in ag_sends:
                rdma.wait_send()

        out_ref[...] = reduced

    out_shape = jax.ShapeDtypeStruct((rows, d_model), jnp.float32)
    vmem = pl.BlockSpec(memory_space=pltpu.VMEM)
    return pl.pallas_call(
        body,
        out_shape=out_shape,
        in_specs=[vmem] * 7,
        out_specs=vmem,
        scratch_shapes=[
            pltpu.VMEM((b, d_model), jnp.bfloat16),
            pltpu.VMEM((N_DEV, rows, d_model), jnp.bfloat16),
            pltpu.VMEM((N_DEV, rows, d_model), jnp.bfloat16),
            pltpu.SemaphoreType.DMA((N_DEV,)),
            pltpu.SemaphoreType.DMA((N_DEV,)),
            pltpu.SemaphoreType.DMA((N_DEV,)),
            pltpu.SemaphoreType.DMA((N_DEV,)),
        ],
    )(x, Win0, Wout0, Win1, Wout1, Win2, Wout2)


# device time: 29940 ns/iter; 1.1525x vs baseline; 1.1378x over previous
import jax
import jax.numpy as jnp
from jax import lax
from jax.experimental import pallas as pl
from jax.experimental.pallas import tpu as pltpu

N_DEV = 8
MASKS_FAR_FIRST = (7, 6, 5, 3, 4, 2, 1)
MASKS_NEAR_FIRST = tuple(reversed(MASKS_FAR_FIRST))


def kernel(x, Win0, Wout0, Win1, Wout1, Win2, Wout2):
    b, d_model = x.shape
    rows = b // N_DEV

    def body(x_ref, win0_ref, wout0_ref, win1_ref, wout1_ref, win2_ref,
             wout2_ref, out_ref, part_ref, rs_ref, ag_ref,
             rs_send_sems, rs_recv_sems, ag_send_sems, ag_recv_sems):
        my = lax.axis_index("i")

        bar = pltpu.get_barrier_semaphore()
        pl.semaphore_signal(bar, inc=1, device_id=(my,),
                            device_id_type=pl.DeviceIdType.MESH)
        pl.semaphore_wait(bar, 1)

        def gray2(t):
            return t ^ (t >> 1)

        my_bits = (my & 4) | gray2(my & 3)

        def peer_at(mask):
            pb = my_bits ^ mask
            return (pb & 4) | gray2(pb & 3)

        def layer_partial(x_bf16, win_ref, wout_ref):
            w_in = win_ref[...].astype(jnp.bfloat16)
            w_out = wout_ref[...].astype(jnp.bfloat16)
            h = jnp.dot(x_bf16, w_in, preferred_element_type=jnp.float32)
            h = jnp.maximum(h, 0.0).astype(jnp.bfloat16)
            return jnp.dot(h, w_out, preferred_element_type=jnp.float32)

        def rs_send(ph, chunk_idx):
            rdma = pltpu.make_async_remote_copy(
                src_ref=part_ref.at[pl.ds(chunk_idx * rows, rows), :],
                dst_ref=rs_ref.at[ph, my],
                send_sem=rs_send_sems.at[chunk_idx],
                recv_sem=rs_recv_sems.at[ph, my],
                device_id=(chunk_idx,),
                device_id_type=pl.DeviceIdType.MESH,
            )
            rdma.start()
            return rdma

        def rs_finish(ph, sends):
            acc = rs_ref[ph, my].astype(jnp.float32)
            for m in MASKS_NEAR_FIRST:
                s = peer_at(m)
                recv = pltpu.make_async_remote_copy(
                    src_ref=rs_ref.at[ph, s],
                    dst_ref=rs_ref.at[ph, s],
                    send_sem=rs_send_sems.at[s],
                    recv_sem=rs_recv_sems.at[ph, s],
                    device_id=(s,),
                    device_id_type=pl.DeviceIdType.MESH,
                )
                recv.wait_recv()
                acc = acc + rs_ref[ph, s].astype(jnp.float32)
            for rdma in sends:
                rdma.wait_send()
            return acc

        part_ref[...] = layer_partial(
            x_ref[...].astype(jnp.bfloat16), win0_ref, wout0_ref
        ).astype(jnp.bfloat16)
        rs_ref[0, my] = part_ref[pl.ds(my * rows, rows), :]
        sends = [rs_send(0, peer_at(m)) for m in MASKS_FAR_FIRST]
        reduced = rs_finish(0, sends)

        for ph, (win_ref, wout_ref) in enumerate(
            [(win1_ref, wout1_ref), (win2_ref, wout2_ref)]
        ):
            ag_ref[ph, my] = reduced.astype(jnp.bfloat16)
            ag_sends = []
            for m in MASKS_FAR_FIRST:
                d = peer_at(m)
                rdma = pltpu.make_async_remote_copy(
                    src_ref=ag_ref.at[ph, my],
                    dst_ref=ag_ref.at[ph, my],
                    send_sem=ag_send_sems.at[d],
                    recv_sem=ag_recv_sems.at[ph, my],
                    device_id=(d,),
                    device_id_type=pl.DeviceIdType.MESH,
                )
                rdma.start()
                ag_sends.append(rdma)

            rs_ref[ph + 1, my] = layer_partial(
                ag_ref[ph, my], win_ref, wout_ref
            ).astype(jnp.bfloat16)

            sends = []
            for m in MASKS_NEAR_FIRST:
                s = peer_at(m)
                recv = pltpu.make_async_remote_copy(
                    src_ref=ag_ref.at[ph, s],
                    dst_ref=ag_ref.at[ph, s],
                    send_sem=ag_send_sems.at[s],
                    recv_sem=ag_recv_sems.at[ph, s],
                    device_id=(s,),
                    device_id_type=pl.DeviceIdType.MESH,
                )
                recv.wait_recv()
                part_ref[pl.ds(s * rows, rows), :] = layer_partial(
                    ag_ref[ph, s], win_ref, wout_ref
                ).astype(jnp.bfloat16)
                sends.append(rs_send(ph + 1, s))
            reduced = rs_finish(ph + 1, sends)
            for rdma in ag_sends:
                rdma.wait_send()

        out_ref[...] = reduced

    out_shape = jax.ShapeDtypeStruct((rows, d_model), jnp.float32)
    vmem = pl.BlockSpec(memory_space=pltpu.VMEM)
    return pl.pallas_call(
        body,
        out_shape=out_shape,
        in_specs=[vmem] * 7,
        out_specs=vmem,
        scratch_shapes=[
            pltpu.VMEM((b, d_model), jnp.bfloat16),
            pltpu.VMEM((3, N_DEV, rows, d_model), jnp.bfloat16),
            pltpu.VMEM((2, N_DEV, rows, d_model), jnp.bfloat16),
            pltpu.SemaphoreType.DMA((N_DEV,)),
            pltpu.SemaphoreType.DMA((3, N_DEV)),
            pltpu.SemaphoreType.DMA((N_DEV,)),
            pltpu.SemaphoreType.DMA((2, N_DEV)),
        ],
        compiler_params=pltpu.CompilerParams(collective_id=0),
    )(x, Win0, Wout0, Win1, Wout1, Win2, Wout2)


# device time: 29794 ns/iter; 1.1582x vs baseline; 1.0049x over previous
import jax
import jax.numpy as jnp
from jax import lax
from jax.experimental import pallas as pl
from jax.experimental.pallas import tpu as pltpu

N_DEV = 8
MASKS_FAR_FIRST = (7, 6, 5, 3, 4, 2, 1)
MASKS_NEAR_FIRST = tuple(reversed(MASKS_FAR_FIRST))


def kernel(x, Win0, Wout0, Win1, Wout1, Win2, Wout2):
    b, d_model = x.shape
    rows = b // N_DEV

    def body(x_ref, win0_ref, wout0_ref, win1_ref, wout1_ref, win2_ref,
             wout2_ref, out_ref, part_ref, far_ref, rs_ref, ag_ref,
             rs_send_sems, rs_recv_sems, ag_send_sems, ag_recv_sems):
        my = lax.axis_index("i")

        bar = pltpu.get_barrier_semaphore()
        pl.semaphore_signal(bar, inc=1, device_id=(my,),
                            device_id_type=pl.DeviceIdType.MESH)
        pl.semaphore_wait(bar, 1)

        def gray2(t):
            return t ^ (t >> 1)

        my_bits = (my & 4) | gray2(my & 3)

        def peer_at(mask):
            pb = my_bits ^ mask
            return (pb & 4) | gray2(pb & 3)

        def layer_partial(x_bf16, win_ref, wout_ref):
            w_in = win_ref[...].astype(jnp.bfloat16)
            w_out = wout_ref[...].astype(jnp.bfloat16)
            h = jnp.dot(x_bf16, w_in, preferred_element_type=jnp.float32)
            h = jnp.maximum(h, 0.0).astype(jnp.bfloat16)
            return jnp.dot(h, w_out, preferred_element_type=jnp.float32)

        def rs_send(ph, chunk_idx, src=None):
            if src is None:
                src = part_ref.at[pl.ds(chunk_idx * rows, rows), :]
            rdma = pltpu.make_async_remote_copy(
                src_ref=src,
                dst_ref=rs_ref.at[ph, my],
                send_sem=rs_send_sems.at[chunk_idx],
                recv_sem=rs_recv_sems.at[ph, my],
                device_id=(chunk_idx,),
                device_id_type=pl.DeviceIdType.MESH,
            )
            rdma.start()
            return rdma

        def rs_finish(ph, sends):
            acc = part_ref[pl.ds(my * rows, rows), :].astype(jnp.float32)
            for m in MASKS_NEAR_FIRST:
                s = peer_at(m)
                recv = pltpu.make_async_remote_copy(
                    src_ref=rs_ref.at[ph, s],
                    dst_ref=rs_ref.at[ph, s],
                    send_sem=rs_send_sems.at[s],
                    recv_sem=rs_recv_sems.at[ph, s],
                    device_id=(s,),
                    device_id_type=pl.DeviceIdType.MESH,
                )
                recv.wait_recv()
                acc = acc + rs_ref[ph, s].astype(jnp.float32)
            for rdma in sends:
                rdma.wait_send()
            return acc

        far = peer_at(7)
        far_ref[...] = layer_partial(
            x_ref[pl.ds(far * rows, rows), :].astype(jnp.bfloat16),
            win0_ref, wout0_ref,
        ).astype(jnp.bfloat16)
        sends = [rs_send(0, far, src=far_ref)]
        part_ref[...] = layer_partial(
            x_ref[...].astype(jnp.bfloat16), win0_ref, wout0_ref
        ).astype(jnp.bfloat16)
        sends += [rs_send(0, peer_at(m)) for m in MASKS_FAR_FIRST[1:]]
        reduced = rs_finish(0, sends)

        for ph, (win_ref, wout_ref) in enumerate(
            [(win1_ref, wout1_ref), (win2_ref, wout2_ref)]
        ):
            ag_ref[ph, my] = reduced.astype(jnp.bfloat16)
            ag_sends = []
            for m in MASKS_FAR_FIRST:
                d = peer_at(m)
                rdma = pltpu.make_async_remote_copy(
                    src_ref=ag_ref.at[ph, my],
                    dst_ref=ag_ref.at[ph, my],
                    send_sem=ag_send_sems.at[d],
                    recv_sem=ag_recv_sems.at[ph, my],
                    device_id=(d,),
                    device_id_type=pl.DeviceIdType.MESH,
                )
                rdma.start()
                ag_sends.append(rdma)

            part_ref[pl.ds(my * rows, rows), :] = layer_partial(
                ag_ref[ph, my], win_ref, wout_ref
            ).astype(jnp.bfloat16)

            sends = []
            for m in MASKS_NEAR_FIRST:
                s = peer_at(m)
                recv = pltpu.make_async_remote_copy(
                    src_ref=ag_ref.at[ph, s],
                    dst_ref=ag_ref.at[ph, s],
                    send_sem=ag_send_sems.at[s],
                    recv_sem=ag_recv_sems.at[ph, s],
                    device_id=(s,),
                    device_id_type=pl.DeviceIdType.MESH,
                )
                recv.wait_recv()
                part_ref[pl.ds(s * rows, rows), :] = layer_partial(
                    ag_ref[ph, s], win_ref, wout_ref
                ).astype(jnp.bfloat16)
                sends.append(rs_send(ph + 1, s))
            reduced = rs_finish(ph + 1, sends)
            for rdma in ag_sends:
                rdma.wait_send()

        out_ref[...] = reduced

    out_shape = jax.ShapeDtypeStruct((rows, d_model), jnp.float32)
    vmem = pl.BlockSpec(memory_space=pltpu.VMEM)
    return pl.pallas_call(
        body,
        out_shape=out_shape,
        in_specs=[vmem] * 7,
        out_specs=vmem,
        scratch_shapes=[
            pltpu.VMEM((b, d_model), jnp.bfloat16),
            pltpu.VMEM((rows, d_model), jnp.bfloat16),
            pltpu.VMEM((3, N_DEV, rows, d_model), jnp.bfloat16),
            pltpu.VMEM((2, N_DEV, rows, d_model), jnp.bfloat16),
            pltpu.SemaphoreType.DMA((N_DEV,)),
            pltpu.SemaphoreType.DMA((3, N_DEV)),
            pltpu.SemaphoreType.DMA((N_DEV,)),
            pltpu.SemaphoreType.DMA((2, N_DEV)),
        ],
        compiler_params=pltpu.CompilerParams(collective_id=0),
    )(x, Win0, Wout0, Win1, Wout1, Win2, Wout2)
